# Initial kernel scaffold; baseline (speedup 1.0000x reference)
import jax
import jax.numpy as jnp
from jax.experimental import pallas as pl


def kernel(P_batch, Q_batch):
    raise NotImplementedError("write your pallas kernel here")


# speedup vs baseline: 10.0398x; 10.0398x over previous
import functools

import jax
import jax.numpy as jnp
import numpy as np
from jax import lax
from jax.experimental import pallas as pl
from jax.experimental.pallas import tpu as pltpu

_N_PROJ = 50
_LPAD = 64
_N = 512
_D = 128
_B = 8


def _projection_weights():
    Z = jax.random.normal(jax.random.key(42), (_N_PROJ, _D, 2), dtype=jnp.float32)
    z0 = Z[:, :, 0]
    z1 = Z[:, :, 1]
    q0 = z0 / jnp.linalg.norm(z0, axis=1, keepdims=True)
    u1 = z1 - jnp.sum(q0 * z1, axis=1, keepdims=True) * q0
    q1 = u1 / jnp.linalg.norm(u1, axis=1, keepdims=True)
    pad = ((0, _LPAD - _N_PROJ), (0, 0))
    return jnp.pad(q0, pad), jnp.pad(q1, pad)


def _bitonic_sort_rows(x):
    n = x.shape[-1]
    lane = lax.broadcasted_iota(jnp.int32, x.shape, x.ndim - 1)
    k = 2
    while k <= n:
        d = k // 2
        while d >= 1:
            low = (lane & d) == 0
            partner = jnp.where(
                low,
                pltpu.roll(x, n - d, axis=x.ndim - 1),
                pltpu.roll(x, d, axis=x.ndim - 1),
            )
            take_min = low != ((lane & k) != 0)
            x = jnp.where(take_min, jnp.minimum(x, partner),
                          jnp.maximum(x, partner))
            d //= 2
        k *= 2
    return x


def _coords_sort_kernel(pt_ref, qt_ref, w0_ref, w1_ref, s_ref):
    W0 = w0_ref[...]
    W1 = w1_ref[...]

    def coords(X):
        c0 = jnp.dot(W0, X, preferred_element_type=jnp.float32)
        c1 = jnp.dot(W1, X, preferred_element_type=jnp.float32)
        th = jnp.arctan2(-c1, -c0)
        return (th + np.pi) * np.float32(0.5 / np.pi)

    S = jnp.concatenate([coords(pt_ref[0]), coords(qt_ref[0])], axis=0)
    s_ref[0] = _bitonic_sort_rows(S)


def _cost_kernel(u_ref, v_ref, out_ref):
    lb = pl.program_id(1)
    srow = lax.broadcasted_iota(jnp.int32, (_N, _N), 0)
    acc = jnp.zeros((), jnp.float32)
    for r in range(8):
        us = u_ref[0, r : r + 1, :]
        vs = v_ref[0, r : r + 1, :]
        VS = jnp.broadcast_to(vs, (_N, _N))
        sh = 1
        while sh < _N:
            VS = jnp.where((srow & sh) != 0, pltpu.roll(VS, _N - sh, axis=1), VS)
            sh *= 2
        t = jnp.abs(us - VS)
        m = jnp.minimum(t, 1.0 - t)
        costs = jnp.sum(m * m, axis=1, keepdims=True)
        w = jnp.min(costs) * np.float32(1.0 / _N)
        l = lb * 8 + r
        acc = acc + jnp.where(l < _N_PROJ, w, 0.0)

    @pl.when(lb == 0)
    def _():
        out_ref[...] = jnp.zeros_like(out_ref)

    out_ref[...] += acc


@jax.jit
def kernel(P_batch, Q_batch):
    W0, W1 = _projection_weights()
    Pt = jnp.transpose(P_batch, (0, 2, 1))
    Qt = jnp.transpose(Q_batch, (0, 2, 1))

    S = pl.pallas_call(
        _coords_sort_kernel,
        grid=(_B,),
        in_specs=[
            pl.BlockSpec((1, _D, _N), lambda i: (i, 0, 0)),
            pl.BlockSpec((1, _D, _N), lambda i: (i, 0, 0)),
            pl.BlockSpec((_LPAD, _D), lambda i: (0, 0)),
            pl.BlockSpec((_LPAD, _D), lambda i: (0, 0)),
        ],
        out_specs=pl.BlockSpec((1, 2 * _LPAD, _N), lambda i: (i, 0, 0)),
        out_shape=jax.ShapeDtypeStruct((_B, 2 * _LPAD, _N), jnp.float32),
    )(Pt, Qt, W0, W1)

    n_blocks = _LPAD // 8
    T = pl.pallas_call(
        _cost_kernel,
        grid=(_B, n_blocks),
        in_specs=[
            pl.BlockSpec((1, 8, _N), lambda i, lb: (i, lb, 0)),
            pl.BlockSpec((1, 8, _N), lambda i, lb: (i, lb + _LPAD // 8, 0)),
        ],
        out_specs=pl.BlockSpec((1, 1, 128), lambda i, lb: (i, 0, 0)),
        out_shape=jax.ShapeDtypeStruct((_B, 1, 128), jnp.float32),
    )(S, S)

    return jnp.mean(jnp.sqrt(T[:, 0, 0] * np.float32(1.0 / _N_PROJ)))


# speedup vs baseline: 28.9402x; 2.8826x over previous
import functools

import jax
import jax.numpy as jnp
import numpy as np
from jax import lax
from jax.experimental import pallas as pl
from jax.experimental.pallas import tpu as pltpu

_N_PROJ = 50
_LPAD = 56
_N = 512
_D = 128
_B = 8


def _projection_weights():
    Z = jax.random.normal(jax.random.key(42), (_N_PROJ, _D, 2), dtype=jnp.float32)
    z0 = Z[:, :, 0]
    z1 = Z[:, :, 1]
    q0 = z0 / jnp.linalg.norm(z0, axis=1, keepdims=True)
    u1 = z1 - jnp.sum(q0 * z1, axis=1, keepdims=True) * q0
    q1 = u1 / jnp.linalg.norm(u1, axis=1, keepdims=True)
    pad = ((0, _LPAD - _N_PROJ), (0, 0))
    return jnp.pad(q0, pad), jnp.pad(q1, pad)


def _bitonic_sort_rows(x, seg):
    w = x.shape[-1]
    lane = lax.broadcasted_iota(jnp.int32, x.shape, x.ndim - 1) & (seg - 1)
    k = 2
    while k <= seg:
        d = k // 2
        while d >= 1:
            low = (lane & d) == 0
            partner = jnp.where(
                low,
                pltpu.roll(x, w - d, axis=x.ndim - 1),
                pltpu.roll(x, d, axis=x.ndim - 1),
            )
            take_min = low != ((lane & k & (seg - 1)) != 0)
            x = jnp.where(take_min, jnp.minimum(x, partner),
                          jnp.maximum(x, partner))
            d //= 2
        k *= 2
    return x


def _coords_sort_kernel(x_ref, w_ref, s_ref):
    W0 = w_ref[0]
    W1 = w_ref[1]
    X = x_ref[0]
    c0 = jnp.dot(W0, X, preferred_element_type=jnp.float32)
    c1 = jnp.dot(W1, X, preferred_element_type=jnp.float32)
    th = jnp.arctan2(-c1, -c0)
    coords = (th + np.pi) * np.float32(0.5 / np.pi)
    s_ref[...] = _bitonic_sort_rows(coords, _N)


_G = 32
_SO = _N // _G
_SPS = 4


def _cost_kernel(*refs):
    u_refs = refs[:_SPS]
    v_refs = refs[_SPS:2 * _SPS]
    out_ref = refs[2 * _SPS]
    lb = pl.program_id(1)
    ng = 8 * _SPS
    rows = ng * _SO
    row = lax.broadcasted_iota(jnp.int32, (rows, _N), 0)
    uv = jnp.concatenate([r[...] for r in u_refs], axis=0)
    vv = jnp.concatenate([r[...] for r in v_refs], axis=0)
    U = jnp.broadcast_to(uv[:, None, :], (ng, _SO, _N)).reshape(rows, _N)
    R = jnp.broadcast_to(vv[:, None, :], (ng, _SO, _N)).reshape(rows, _N)
    m = 1
    while m < _SO:
        R = jnp.where((row & m) != 0, pltpu.roll(R, _N - _G * m, axis=1), R)
        m *= 2
    best = None
    for si in range(_G):
        plane = pltpu.roll(R, _N - si, axis=1) if si else R
        t = U - plane
        dc = t - jnp.round(t)
        cs = jnp.sum(dc * dc, axis=1, keepdims=True)
        best = cs if best is None else jnp.minimum(best, cs)
    wg = jnp.min(best.reshape(ng, _SO), axis=1) * np.float32(1.0 / _N)
    l8 = lb * 8 + jnp.arange(8)
    mask = jnp.tile((l8 < _N_PROJ).astype(jnp.float32), _SPS)
    accs = jnp.sum((wg * mask).reshape(_SPS, 8), axis=1).reshape(_SPS, 1, 1)

    @pl.when(lb == 0)
    def _():
        out_ref[...] = jnp.zeros_like(out_ref)

    out_ref[...] += accs


@jax.jit
def kernel(P_batch, Q_batch):
    W0, W1 = _projection_weights()
    P_all = jnp.transpose(P_batch, (2, 0, 1)).reshape(_D, _B * _N)
    Q_all = jnp.transpose(Q_batch, (2, 0, 1)).reshape(_D, _B * _N)

    XPQ = jnp.stack([P_all, Q_all])
    WS = jnp.stack([W0, W1])
    lanes_half = _B * _N // 2
    S = pl.pallas_call(
        _coords_sort_kernel,
        grid=(2, 2),
        in_specs=[
            pl.BlockSpec((1, _D, lanes_half), lambda g, h: (g, 0, h)),
            pl.BlockSpec((2, _LPAD, _D), lambda g, h: (0, 0, 0)),
        ],
        out_specs=pl.BlockSpec((_LPAD, lanes_half), lambda g, h: (g, h)),
        out_shape=jax.ShapeDtypeStruct((2 * _LPAD, _B * _N), jnp.float32),
    )(XPQ, WS)

    n_blocks = _LPAD // 8
    u_specs = [
        pl.BlockSpec((8, _N), functools.partial(
            lambda s, i, lb: (lb, _SPS * i + s), s))
        for s in range(_SPS)
    ]
    v_specs = [
        pl.BlockSpec((8, _N), functools.partial(
            lambda s, i, lb: (lb + n_blocks, _SPS * i + s), s))
        for s in range(_SPS)
    ]
    T = pl.pallas_call(
        _cost_kernel,
        grid=(_B // _SPS, n_blocks),
        in_specs=u_specs + v_specs,
        out_specs=pl.BlockSpec((_SPS, 1, 128), lambda i, lb: (i, 0, 0)),
        out_shape=jax.ShapeDtypeStruct((_B, 1, 128), jnp.float32),
    )(*([S] * (2 * _SPS)))

    return jnp.mean(jnp.sqrt(T[:, 0, 0] * np.float32(1.0 / _N_PROJ)))


# speedup vs baseline: 54.0468x; 1.8675x over previous
import jax
import jax.numpy as jnp
import numpy as np
from jax import lax
from jax.experimental import pallas as pl
from jax.experimental.pallas import tpu as pltpu

_N_PROJ = 50
_LPAD = 56
_N = 512
_D = 128
_B = 8


def _projection_weights():
    Z = jax.random.normal(jax.random.key(42), (_N_PROJ, _D, 2), dtype=jnp.float32)
    z0 = Z[:, :, 0]
    z1 = Z[:, :, 1]
    q0 = z0 / jnp.linalg.norm(z0, axis=1, keepdims=True)
    u1 = z1 - jnp.sum(q0 * z1, axis=1, keepdims=True) * q0
    q1 = u1 / jnp.linalg.norm(u1, axis=1, keepdims=True)
    pad = ((0, _LPAD - _N_PROJ), (0, 0))
    return jnp.stack([jnp.pad(q0, pad), jnp.pad(q1, pad)])


def _bitonic_sort_rows(x, seg):
    w = x.shape[-1]
    ax = x.ndim - 1
    lane = lax.broadcasted_iota(jnp.int32, x.shape, ax) & (seg - 1)

    def sign(k):
        return (lane & k & (seg - 1)) != 0

    def signvec(mask):
        return jnp.where(mask, np.float32(-1.0), np.float32(1.0))

    y = x * signvec(sign(2))
    k = 2
    while k <= seg:
        d = k // 2
        while d >= 1:
            low = (lane & d) == 0
            a = pltpu.roll(y, w - d, axis=ax)
            b = pltpu.roll(y, d, axis=ax)
            y = jnp.where(low, jnp.minimum(y, a), jnp.maximum(y, b))
            d //= 2
        prev, k = sign(k), k * 2
        nxt = sign(k)
        if k <= seg:
            y = y * signvec(prev != nxt)
    return y


_DN = (((1,), (1,)), ((), ()))


def _coords_sort(XP, XQ, W0, W1):
    coords = []
    for X in (XP, XQ):
        c0 = lax.dot_general(W0, X, _DN, preferred_element_type=jnp.float32)
        c1 = lax.dot_general(W1, X, _DN, preferred_element_type=jnp.float32)
        th = jnp.arctan2(-c1, -c0)
        coords.append((th + np.pi) * np.float32(0.5 / np.pi))
    return _bitonic_sort_rows(jnp.concatenate(coords, axis=0), _N)


_G = 32
_SO = _N // _G
_NV = 4


def _cost_block(uv, vv):
    ng = uv.shape[0]
    rows = ng * _SO
    row = lax.broadcasted_iota(jnp.int32, (rows, _N), 0)
    R = jnp.broadcast_to(vv[:, None, :], (ng, _SO, _N)).reshape(rows, _N)
    m = 1
    while m < _SO:
        R = jnp.where((row & m) != 0, pltpu.roll(R, _N - _G * m, axis=1), R)
        m *= 2
    R3 = R.reshape(ng, _SO, _N)
    uvs = [uv] + [pltpu.roll(uv, _N - v, axis=1) for v in range(1, _NV)]
    ones = jnp.ones((_N, 1), jnp.float32)
    best = None
    for rho in range(0, _G, _NV):
        Rr = pltpu.roll(R3, _N - rho, axis=2) if rho else R3
        for v in range(_NV):
            t = uvs[v][:, None, :] - Rr
            dc = t - jnp.round(t)
            dc2 = (dc * dc).reshape(rows, _N)
            cs = jnp.dot(dc2, ones, preferred_element_type=jnp.float32)
            best = cs if best is None else jnp.minimum(best, cs)
    return jnp.min(best.reshape(ng, _SO), axis=1) * np.float32(1.0 / _N)


_NB = _LPAD // 8


def _fused_kernel(p_ref, q_ref, w_ref, out_ref, s_vm):
    step = pl.program_id(0)

    @pl.when(step < 2)
    def _():
        s_vm[step] = _coords_sort(p_ref[...], q_ref[...], w_ref[0], w_ref[1])

    @pl.when(step >= 2)
    def _():
        lb = step - 2
        uvs, vvs = [], []
        for h in range(2):
            us = s_vm[h, pl.ds(lb * 8, 8), :]
            vs = s_vm[h, pl.ds(_LPAD + lb * 8, 8), :]
            uvs += [us[:, s * _N:(s + 1) * _N] for s in range(4)]
            vvs += [vs[:, s * _N:(s + 1) * _N] for s in range(4)]
        wg = _cost_block(jnp.concatenate(uvs, axis=0),
                         jnp.concatenate(vvs, axis=0))
        l8 = lb * 8 + jnp.arange(8)
        mask = jnp.tile((l8 < _N_PROJ).astype(jnp.float32), _B)
        accs = jnp.sum((wg * mask).reshape(_B, 8), axis=1).reshape(_B, 1, 1)

        @pl.when(lb == 0)
        def _():
            out_ref[...] = jnp.zeros_like(out_ref)

        out_ref[...] += accs


@jax.jit
def kernel(P_batch, Q_batch):
    P_flat = P_batch.reshape(_B * _N, _D)
    Q_flat = Q_batch.reshape(_B * _N, _D)
    WS = _projection_weights()
    half = _B * _N // 2
    T = pl.pallas_call(
        _fused_kernel,
        grid=(2 + _NB,),
        in_specs=[
            pl.BlockSpec((half, _D), lambda s: (jnp.minimum(s, 1), 0)),
            pl.BlockSpec((half, _D), lambda s: (jnp.minimum(s, 1), 0)),
            pl.BlockSpec((2, _LPAD, _D), lambda s: (0, 0, 0)),
        ],
        out_specs=pl.BlockSpec((_B, 1, 128), lambda s: (0, 0, 0)),
        out_shape=jax.ShapeDtypeStruct((_B, 1, 128), jnp.float32),
        scratch_shapes=[
            pltpu.VMEM((2, 2 * _LPAD, half), jnp.float32)],
    )(P_flat, Q_flat, WS)

    return jnp.mean(jnp.sqrt(T[:, 0, 0] * np.float32(1.0 / _N_PROJ)))
